# baseline (speedup 1.0000x reference)
import functools

import jax
import jax.numpy as jnp
from jax import lax
from jax.experimental import pallas as pl
from jax.experimental.pallas import tpu as pltpu
from jax.experimental.pallas import tpu_sc as plsc

NC, NS = 2, 16
NW = NC * NS
BATCH = 4096
SEQ = 200
D = 32
VOCAB = 1000000
VT = VOCAB // 128
VT_PER_W = VT // NW + 1
BB = BATCH // 128
NBLK = SEQ * BB
BLK_W = NBLK // NW


def _repack_body(wt_hbm, tbl_hbm, blk_v, row_v, sem):
    wid = lax.axis_index("s") * NC + lax.axis_index("c")
    i16 = lax.iota(jnp.int32, 16)
    es_vec = jnp.bitwise_and(i16, 7)
    e_lo = jnp.right_shift(i16, 3)

    def make_r_body(nrows):
        def r_body(r, c2):
            for k in range(8):
                ev = e_lo + 2 * (k & 1)
                vs = jnp.full((16,), 0, jnp.int32) + (4 * r + (k >> 1))
                x = plsc.load_gather(blk_v, [ev, es_vec, vs])
                row_v[r, pl.ds(16 * k, 16)] = x
            return c2
        return r_body

    def tile_body(j, carry):
        vt = j * NW + wid

        @pl.when(vt < VT)
        def _full():
            for E in range(4):
                pltpu.sync_copy(
                    wt_hbm.at[pl.ds(8 * E, 8), pl.ds(vt * 128, 128)],
                    blk_v.at[E])
            lax.fori_loop(0, 32, make_r_body(32), 0)
            pltpu.sync_copy(row_v, tbl_hbm.at[pl.ds(vt * 32, 32)])

        @pl.when(vt == VT)
        def _partial():
            for E in range(4):
                pltpu.sync_copy(
                    wt_hbm.at[pl.ds(8 * E, 8), pl.ds(vt * 128, 64)],
                    blk_v.at[E, :, pl.ds(0, 64)])
            lax.fori_loop(0, 16, make_r_body(16), 0)
            pltpu.sync_copy(row_v.at[pl.ds(0, 16)],
                            tbl_hbm.at[pl.ds(vt * 32, 16)])

        return carry

    lax.fori_loop(0, VT_PER_W, tile_body, 0)


def _lookup_body(idx_hbm, tbl_hbm, pos_hbm, out_hbm,
                 idx_v, rows_v, t_v, pos_v, sem):
    wid = lax.axis_index("s") * NC + lax.axis_index("c")
    pltpu.sync_copy(pos_hbm, pos_v)
    i16 = lax.iota(jnp.int32, 16)
    es_vec = jnp.bitwise_and(i16, 7)
    e_hi = jnp.right_shift(i16, 3)

    def blk_body(j, carry):
        bid = wid * BLK_W + j
        s = bid // BB
        b = bid % BB
        pltpu.sync_copy(idx_hbm.at[s, pl.ds(b * 128, 128)], idx_v)
        pltpu.async_copy(tbl_hbm.at[idx_v], rows_v, sem).wait()
        p0 = pos_v[s, 0:16]
        p1 = pos_v[s, 16:32]

        def t_body(t, c2):
            x0 = rows_v[t, 0:16] + p0
            x1 = rows_v[t, 16:32] + p1
            bvec = jnp.full((16,), 0, jnp.int32) + t
            plsc.store_scatter(t_v, [e_hi, es_vec, bvec], x0)
            plsc.store_scatter(t_v, [2 + e_hi, es_vec, bvec], x1)
            return c2

        lax.fori_loop(0, 128, t_body, 0)
        pltpu.sync_copy(t_v, out_hbm.at[s, :, b])
        return carry

    lax.fori_loop(0, BLK_W, blk_body, 0)


def kernel(inputs, word_table, pos_table):
    mesh = plsc.VectorSubcoreMesh(core_axis_name="c", subcore_axis_name="s")
    k1 = pl.kernel(
        _repack_body,
        out_type=jax.ShapeDtypeStruct((VOCAB // 4, 128), jnp.float32),
        mesh=mesh,
        scratch_types=[
            pltpu.VMEM((4, 8, 128), jnp.float32),
            pltpu.VMEM((32, 128), jnp.float32),
            pltpu.SemaphoreType.DMA,
        ],
        compiler_params=pltpu.CompilerParams(
            use_tc_tiling_on_sc=True, needs_layout_passes=False),
    )
    tbl = k1(word_table.T).reshape(VOCAB, D)

    k2 = pl.kernel(
        _lookup_body,
        out_type=jax.ShapeDtypeStruct((SEQ, 4, BB, 8, 128), jnp.float32),
        mesh=mesh,
        scratch_types=[
            pltpu.VMEM((128,), jnp.int32),
            pltpu.VMEM((128, D), jnp.float32),
            pltpu.VMEM((4, 8, 128), jnp.float32),
            pltpu.VMEM((SEQ, D), jnp.float32),
            pltpu.SemaphoreType.DMA,
        ],
        compiler_params=pltpu.CompilerParams(
            use_tc_tiling_on_sc=False, needs_layout_passes=False),
    )
    out5 = k2(inputs.T.astype(jnp.int32), tbl, pos_table)
    return out5.transpose(2, 4, 0, 1, 3).reshape(BATCH, SEQ, D)


# speedup vs baseline: 11.6589x; 11.6589x over previous
import functools

import jax
import jax.numpy as jnp
from jax import lax
from jax.experimental import pallas as pl
from jax.experimental.pallas import tpu as pltpu
from jax.experimental.pallas import tpu_sc as plsc

NC, NS = 2, 16
NW = NC * NS
BATCH = 4096
SEQ = 200
D = 32
VOCAB = 1000000
VT = VOCAB // 128
BT = 4
NBATCH = VT // BT
BATCH_IT = NBATCH // NW + 1
GB = 512
NG = BATCH * SEQ // GB
NG_W = NG // NW
BQ = BATCH // GB


def _repack_body(wt_hbm, tail_hbm, tbl_hbm, blk_v, row_v, tail_v, sems):
    wid = lax.axis_index("s") * NC + lax.axis_index("c")
    i16 = lax.iota(jnp.int32, 16)
    es_vec = jnp.bitwise_and(i16, 7)
    e_lo = jnp.right_shift(i16, 3)

    def load_batch(bt, p):
        for E in range(4):
            pltpu.async_copy(
                wt_hbm.at[pl.ds(8 * E, 8), pl.ds(bt * 128 * BT, 128 * BT)],
                blk_v.at[p, E], sems.at[p])

    def wait_batch(p):
        for E in range(4):
            pltpu.make_async_copy(
                wt_hbm.at[pl.ds(0, 8), pl.ds(0, 128 * BT)],
                blk_v.at[p, 0], sems.at[p]).wait()

    def transform(p, nrows):
        @functools.partial(plsc.parallel_loop, 0, nrows, unroll=2)
        def _(r):
            base = (r >> 5) * 128 + 4 * (r & 31)
            for k in range(8):
                ev = e_lo + 2 * (k & 1)
                vs = jnp.full((16,), 0, jnp.int32) + (base + (k >> 1))
                x = plsc.load_gather(blk_v.at[p], [ev, es_vec, vs])
                row_v[p, r, pl.ds(16 * k, 16)] = x

    def store_batch(bt, p, nrows):
        pltpu.async_copy(row_v.at[p, pl.ds(0, nrows)],
                         tbl_hbm.at[pl.ds(bt * 32 * BT, nrows)],
                         sems.at[2 + p])

    def wait_store(p):
        pltpu.make_async_copy(row_v.at[p], tbl_hbm.at[pl.ds(0, 32 * BT)],
                              sems.at[2 + p]).wait()

    for p in range(2):
        bt0 = p * NW + wid

        @pl.when(bt0 < NBATCH)
        def _(bt0=bt0, p=p):
            load_batch(bt0, p)

    def it_body(j, carry):
        for p in range(2):
            bt = (2 * j + p) * NW + wid
            nxt = bt + 2 * NW

            @pl.when(bt < NBATCH)
            def _(bt=bt, p=p, nxt=nxt):
                wait_batch(p)

                @pl.when(jnp.int32(2 * j + p) >= 2)
                def _():
                    wait_store(p)

                transform(p, 32 * BT)
                store_batch(bt, p, 32 * BT)

                @pl.when(nxt < NBATCH)
                def _():
                    load_batch(nxt, p)

        return carry

    lax.fori_loop(0, BATCH_IT // 2 + 1, it_body, 0)

    for p in range(2):
        wait_store(p)

    @pl.when(wid == 0)
    def _tail():
        pltpu.sync_copy(tail_hbm, tail_v)
        pltpu.sync_copy(tail_v, tbl_hbm.at[pl.ds(VT * 32, 16)])


def _lookup_body(idx_hbm, tbl_hbm, pos_hbm, out_hbm,
                 idx_v, rows_v, t_v, pos_v, gsem, wsem):
    wid = lax.axis_index("s") * NC + lax.axis_index("c")
    pltpu.sync_copy(pos_hbm, pos_v)
    i16 = lax.iota(jnp.int32, 16)
    es_vec = jnp.bitwise_and(i16, 7)
    e_hi = jnp.right_shift(i16, 3)

    def issue(gid, p):
        s = gid // BQ
        q = gid % BQ
        pltpu.sync_copy(idx_hbm.at[s, pl.ds(q * GB, GB)], idx_v.at[p])
        pltpu.async_copy(tbl_hbm.at[idx_v.at[p]], rows_v.at[p], gsem.at[p])

    def wait_gather(p):
        pltpu.make_async_copy(tbl_hbm.at[idx_v.at[p]], rows_v.at[p],
                              gsem.at[p]).wait()

    for p in range(2):
        issue(wid * NG_W + p, p)

    def g_body(g, carry):
        for p in range(2):
            j = 2 * g + p
            gid = wid * NG_W + j
            s = gid // BQ
            q = gid % BQ
            wait_gather(p)

            @pl.when(jnp.int32(j) >= 2)
            def _():
                pltpu.make_async_copy(
                    t_v.at[p], out_hbm.at[0, :, pl.ds(0, 4)],
                    wsem.at[p]).wait()

            p0 = pos_v[s, 0:16]
            p1 = pos_v[s, 16:32]

            @functools.partial(plsc.parallel_loop, 0, GB, unroll=2)
            def _(t):
                x0 = rows_v[p, t, 0:16] + p0
                x1 = rows_v[p, t, 16:32] + p1
                bq_i = jnp.full((16,), 0, jnp.int32) + (t >> 7)
                bs_i = jnp.full((16,), 0, jnp.int32) + (t & 127)
                plsc.store_scatter(t_v.at[p], [e_hi, bq_i, es_vec, bs_i], x0)
                plsc.store_scatter(t_v.at[p], [2 + e_hi, bq_i, es_vec, bs_i],
                                   x1)

            pltpu.async_copy(t_v.at[p], out_hbm.at[s, :, pl.ds(q * 4, 4)],
                             wsem.at[p])

            @pl.when(jnp.int32(j) + 2 < NG_W)
            def _():
                issue(gid + 2, p)

        return carry

    lax.fori_loop(0, NG_W // 2, g_body, 0)
    for p in range(2):
        pltpu.make_async_copy(t_v.at[p], out_hbm.at[0, :, pl.ds(0, 4)],
                              wsem.at[p]).wait()


def kernel(inputs, word_table, pos_table):
    mesh = plsc.VectorSubcoreMesh(core_axis_name="c", subcore_axis_name="s")
    k1 = pl.kernel(
        _repack_body,
        out_type=jax.ShapeDtypeStruct((VOCAB // 4, 128), jnp.float32),
        mesh=mesh,
        scratch_types=[
            pltpu.VMEM((2, 4, 8, 128 * BT), jnp.float32),
            pltpu.VMEM((2, 32 * BT, 128), jnp.float32),
            pltpu.VMEM((16, 128), jnp.float32),
            pltpu.SemaphoreType.DMA((4,)),
        ],
        compiler_params=pltpu.CompilerParams(
            use_tc_tiling_on_sc=True, needs_layout_passes=False),
    )
    tail = lax.slice(word_table, (VT * 128, 0), (VOCAB, D)).reshape(16, 128)
    tbl = k1(word_table.T, tail).reshape(VOCAB, D)

    k2 = pl.kernel(
        _lookup_body,
        out_type=jax.ShapeDtypeStruct((SEQ, 4, BATCH // 128, 8, 128),
                                      jnp.float32),
        mesh=mesh,
        scratch_types=[
            pltpu.VMEM((2, GB), jnp.int32),
            pltpu.VMEM((2, GB, D), jnp.float32),
            pltpu.VMEM((2, 4, 4, 8, 128), jnp.float32),
            pltpu.VMEM((SEQ, D), jnp.float32),
            pltpu.SemaphoreType.DMA((2,)),
            pltpu.SemaphoreType.DMA((2,)),
        ],
        compiler_params=pltpu.CompilerParams(
            use_tc_tiling_on_sc=False, needs_layout_passes=False),
    )
    out5 = k2(inputs.T.astype(jnp.int32), tbl, pos_table)
    return out5.transpose(2, 4, 0, 1, 3).reshape(BATCH, SEQ, D)
